# baseline (speedup 1.0000x reference)
import functools
import math

import jax
import jax.numpy as jnp
from jax import lax
from jax.experimental import pallas as pl
from jax.experimental.pallas import tpu as pltpu
from jax.experimental.pallas import tpu_sc as plsc

N = 10000
E = 320000
D = 128
NC = 2
NS = 16
NW = NC * NS
EPT = E // NW
CHUNK = 128
NCHUNK = 80
EPAD = NCHUNK * CHUNK - EPT
NBUF = 4
NACC = 10008
DH = D // 2
RPS = 624
TAIL = N - NS * RPS

_INV = 1.0 / math.sqrt(1.0 + 1e-5)


def _segment_sum_sc(hL, hR, src3, dst3, zeros):
    mesh = plsc.VectorSubcoreMesh(
        core_axis_name="c", subcore_axis_name="s", num_cores=NC, num_subcores=NS
    )

    @functools.partial(
        pl.kernel,
        out_type=jax.ShapeDtypeStruct((2 * NC, N, DH), jnp.float32),
        mesh=mesh,
        compiler_params=pltpu.CompilerParams(use_tc_tiling_on_sc=False),
        scratch_types=[
            pltpu.VMEM((NCHUNK, CHUNK), jnp.int32),
            pltpu.VMEM((NCHUNK, CHUNK), jnp.int32),
            [pltpu.VMEM((CHUNK, DH), jnp.float32) for _ in range(NBUF)],
            pltpu.VMEM_SHARED((NACC, DH), jnp.float32),
            [pltpu.SemaphoreType.DMA for _ in range(NBUF)],
            [pltpu.SemaphoreType.DMA for _ in range(NBUF)],
        ],
    )
    def k(hL_hbm, hR_hbm, src_hbm, dst_hbm, z_hbm, out_hbm, srcv, dstv, rows,
          acc, gsem, ssem):
        cid = lax.axis_index("c")
        sid = lax.axis_index("s")
        wid = sid * NC + cid
        r0 = sid * RPS

        pltpu.sync_copy(src_hbm.at[pl.ds(wid * NCHUNK, NCHUNK), :], srcv)
        pltpu.sync_copy(dst_hbm.at[pl.ds(wid * NCHUNK, NCHUNK), :], dstv)

        def run_pass(h_hbm, half):
            pltpu.sync_copy(z_hbm.at[pl.ds(r0, RPS)], acc.at[pl.ds(r0, RPS)])

            @pl.when(sid == NS - 1)
            def _():
                pltpu.sync_copy(z_hbm.at[pl.ds(NS * RPS, TAIL)],
                                acc.at[pl.ds(NS * RPS, TAIL)])

            plsc.subcore_barrier()

            def gather(i, b):
                return pltpu.make_async_copy(h_hbm.at[srcv.at[i]], rows[b],
                                             gsem[b])

            def scatter(i, b):
                return pltpu.make_async_copy(rows[b], acc.at[dstv.at[i]],
                                             ssem[b])

            for b in range(NBUF):
                gather(b, b).start()

            @pl.loop(0, (NCHUNK - NBUF) // NBUF)
            def _(j):
                i0 = j * NBUF
                for b in range(NBUF):
                    gather(i0 + b, b).wait()
                    scatter(i0 + b, b).start(add=True)
                for b in range(NBUF):
                    scatter(i0 + b, b).wait()
                    gather(i0 + NBUF + b, b).start()

            i0 = NCHUNK - NBUF
            for b in range(NBUF):
                gather(i0 + b, b).wait()
                scatter(i0 + b, b).start(add=True)
            for b in range(NBUF):
                scatter(i0 + b, b).wait()

            plsc.subcore_barrier()
            out_slot = out_hbm.at[cid * 2 + half]
            pltpu.sync_copy(acc.at[pl.ds(r0, RPS)], out_slot.at[pl.ds(r0, RPS)])

            @pl.when(sid == NS - 1)
            def _():
                pltpu.sync_copy(acc.at[pl.ds(NS * RPS, TAIL)],
                                out_slot.at[pl.ds(NS * RPS, TAIL)])

        run_pass(hL_hbm, 0)
        run_pass(hR_hbm, 1)

    return k(hL, hR, src3, dst3, zeros)


def _sum_parts(x_ref, p00_ref, p01_ref, p10_ref, p11_ref):
    return x_ref[...] + jnp.concatenate(
        [p00_ref[...] + p10_ref[...], p01_ref[...] + p11_ref[...]], axis=1)


def _mlp0_body(x_ref, p00_ref, p01_ref, p10_ref, p11_ref,
               w1t_ref, b1_ref, g1_ref, be1_ref,
               w2t_ref, b2_ref, g0_ref, be0_ref, o_ref):
    t = _sum_parts(x_ref, p00_ref, p01_ref, p10_ref, p11_ref)
    a = jnp.dot(t, w1t_ref[...], preferred_element_type=jnp.float32) + b1_ref[...]
    a = jnp.maximum(a * (_INV * g1_ref[...]) + be1_ref[...], 0.0)
    h = jnp.dot(a, w2t_ref[...], preferred_element_type=jnp.float32) + b2_ref[...]
    o_ref[...] = jnp.maximum(h * (_INV * g0_ref[...]) + be0_ref[...], 0.0)


def _head_body(h_ref, p00_ref, p01_ref, p10_ref, p11_ref,
               w1t_ref, b1_ref, g1_ref, be1_ref,
               w2t_ref, b2_ref, wct_ref, bc_ref, wst_ref, bs_ref,
               wmt_ref, bm_ref, main_ref, sim_ref, he_ref):
    t = _sum_parts(h_ref, p00_ref, p01_ref, p10_ref, p11_ref)
    a = jnp.dot(t, w1t_ref[...], preferred_element_type=jnp.float32) + b1_ref[...]
    a = jnp.maximum(a * (_INV * g1_ref[...]) + be1_ref[...], 0.0)
    h2 = jnp.dot(a, w2t_ref[...], preferred_element_type=jnp.float32) + b2_ref[...]

    main = jnp.dot(h2, wct_ref[...], preferred_element_type=jnp.float32) + bc_ref[...]
    m = jnp.max(main, axis=-1, keepdims=True)
    s = main - m
    main_ref[...] = s - jnp.log(jnp.sum(jnp.exp(s), axis=-1, keepdims=True))

    sim = jnp.dot(h2, wst_ref[...], preferred_element_type=jnp.float32) + bs_ref[...]
    ms = jnp.max(sim, axis=-1, keepdims=True)
    es = jnp.exp(sim - ms)
    sim_ref[...] = es / jnp.sum(es, axis=-1, keepdims=True)

    he = jnp.dot(h2, wmt_ref[...], preferred_element_type=jnp.float32) + bm_ref[...]
    he_ref[...] = 1.0 / (1.0 + jnp.exp(-he))


_BM = 1000


def _row(i):
    return (i, 0)


def _fixed(i):
    return (0, 0)


def _mlp0(x, parts, w1t, b1, g1, be1, w2t, b2, g0, be0):
    rspec = pl.BlockSpec((_BM, D), _row)
    pspec = pl.BlockSpec((_BM, DH), _row)
    wspec = pl.BlockSpec((D, D), _fixed)
    vspec = pl.BlockSpec((1, D), _fixed)
    return pl.pallas_call(
        _mlp0_body,
        out_shape=jax.ShapeDtypeStruct((N, D), jnp.float32),
        grid=(N // _BM,),
        in_specs=[rspec, pspec, pspec, pspec, pspec,
                  wspec, vspec, vspec, vspec,
                  wspec, vspec, vspec, vspec],
        out_specs=rspec,
    )(x, parts[0], parts[1], parts[2], parts[3],
      w1t, b1, g1, be1, w2t, b2, g0, be0)


def _heads(h, parts, w1t, b1, g1, be1, w2t, b2, wct, bc, wst, bs, wmt, bm):
    rspec = pl.BlockSpec((_BM, D), _row)
    pspec = pl.BlockSpec((_BM, DH), _row)
    wspec = pl.BlockSpec((D, D), _fixed)
    vspec = pl.BlockSpec((1, D), _fixed)
    return pl.pallas_call(
        _head_body,
        out_shape=(
            jax.ShapeDtypeStruct((N, 40), jnp.float32),
            jax.ShapeDtypeStruct((N, 40), jnp.float32),
            jax.ShapeDtypeStruct((N, 2), jnp.float32),
        ),
        grid=(N // _BM,),
        in_specs=[rspec, pspec, pspec, pspec, pspec,
                  wspec, vspec, vspec, vspec,
                  wspec, vspec,
                  pl.BlockSpec((D, 40), _fixed), pl.BlockSpec((1, 40), _fixed),
                  pl.BlockSpec((D, 40), _fixed), pl.BlockSpec((1, 40), _fixed),
                  pl.BlockSpec((D, 2), _fixed), pl.BlockSpec((1, 2), _fixed)],
        out_specs=(
            pl.BlockSpec((_BM, 40), _row),
            pl.BlockSpec((_BM, 40), _row),
            pl.BlockSpec((_BM, 2), _row),
        ),
    )(h, parts[0], parts[1], parts[2], parts[3],
      w1t, b1, g1, be1, w2t, b2, wct, bc, wst, bs, wmt, bm)


def kernel(x, edge_index, params):
    src = edge_index[0].astype(jnp.int32)
    dst = edge_index[1].astype(jnp.int32)
    src3 = jnp.pad(src.reshape(NW, EPT), ((0, 0), (0, EPAD)),
                   constant_values=0).reshape(NW * NCHUNK, CHUNK)
    dst3 = jnp.pad(dst.reshape(NW, EPT), ((0, 0), (0, EPAD)),
                   constant_values=N).reshape(NW * NCHUNK, CHUNK)
    zeros = jnp.zeros((N, DH), jnp.float32)

    c0, c1 = params["conv0"], params["conv1"]

    def vec(v):
        return v.reshape(1, -1)

    parts0 = _segment_sum_sc(x[:, :DH], x[:, DH:], src3, dst3, zeros)
    h1 = _mlp0(
        x, parts0,
        c0["lin1"]["W"].T, vec(c0["lin1"]["b"]), vec(c0["bn"]["g"]), vec(c0["bn"]["be"]),
        c0["lin2"]["W"].T, vec(c0["lin2"]["b"]),
        vec(params["bn0"]["g"]), vec(params["bn0"]["be"]),
    )

    parts1 = _segment_sum_sc(h1[:, :DH], h1[:, DH:], src3, dst3, zeros)
    wmt = jnp.concatenate([params["homo"]["W"].T, params["ent"]["W"].T], axis=1)
    bm = jnp.concatenate([params["homo"]["b"], params["ent"]["b"]]).reshape(1, 2)
    main, sim, he = _heads(
        h1, parts1,
        c1["lin1"]["W"].T, vec(c1["lin1"]["b"]), vec(c1["bn"]["g"]), vec(c1["bn"]["be"]),
        c1["lin2"]["W"].T, vec(c1["lin2"]["b"]),
        params["cls"]["W"].T, vec(params["cls"]["b"]),
        params["sim"]["W"].T, vec(params["sim"]["b"]),
        wmt, bm,
    )
    return main, sim, he[:, 0], he[:, 1]
